# baseline (speedup 1.0000x reference)
import jax
import jax.numpy as jnp
from jax import lax
from jax.experimental import pallas as pl
from jax.experimental.pallas import tpu as pltpu
from jax.experimental.pallas import tpu_sc as plsc

R, N = 128, 32768
NC, NS, L = 2, 16, 16
NW = NC * NS
RPW = R // NW
U = 8
SLICES = N // L


def _argmax_body(x_hbm, out_hbm, buf0, buf1, res_v, sem0, sem1):
    wid = lax.axis_index("s") * NC + lax.axis_index("c")
    row0 = wid * RPW
    bufs = (buf0, buf1)
    sems = (sem0, sem1)
    lanes = lax.iota(jnp.int32, L)

    pltpu.make_async_copy(x_hbm.at[row0], buf0, sem0).start()

    res = jnp.zeros((L,), jnp.int32)
    for t in range(RPW):
        buf, sem = bufs[t % 2], sems[t % 2]
        if t + 1 < RPW:
            nxt = (t + 1) % 2
            pltpu.make_async_copy(x_hbm.at[row0 + t + 1], bufs[nxt], sems[nxt]).start()
        pltpu.make_async_copy(x_hbm.at[row0 + t], buf, sem).wait()

        def inner(i, carry):
            vmax, vit = carry
            for u in range(U):
                it = i * U + u
                x = buf[pl.ds(it * L, L)]
                upd = x > vmax
                vmax = jnp.maximum(x, vmax)
                vit = jnp.where(upd, it, vit)
            return vmax, vit

        vmax0 = jnp.full((L,), -jnp.inf, jnp.float32)
        vit0 = jnp.zeros((L,), jnp.int32)
        vmax, vit = lax.fori_loop(0, SLICES // U, inner, (vmax0, vit0))

        m = jnp.max(vmax)
        g = vit * L + lanes
        cand = jnp.where(vmax == m, g, jnp.int32(2**31 - 1))
        idx = jnp.min(cand)
        res = jnp.where(lanes == t, idx, res)

    res_v[...] = res
    pltpu.sync_copy(res_v, out_hbm.at[wid])


_call = pl.kernel(
    _argmax_body,
    mesh=plsc.VectorSubcoreMesh(core_axis_name="c", subcore_axis_name="s"),
    compiler_params=pltpu.CompilerParams(needs_layout_passes=False),
    out_type=jax.ShapeDtypeStruct((NW, L), jnp.int32),
    scratch_types=[
        pltpu.VMEM((N,), jnp.float32),
        pltpu.VMEM((N,), jnp.float32),
        pltpu.VMEM((L,), jnp.int32),
        pltpu.SemaphoreType.DMA,
        pltpu.SemaphoreType.DMA,
    ],
)


@jax.jit
def kernel(inputs):
    out = _call(inputs)
    return out[:, :RPW].reshape(R)
